# baseline (speedup 1.0000x reference)
import numpy as np
import jax
import jax.numpy as jnp
from jax import lax
from jax.experimental import pallas as pl
from jax.experimental.pallas import tpu as pltpu
from jax.experimental.pallas import tpu_sc as plsc

_POOLING_COUNTS = [15, 8, 8]
_N = 64
_B = 32
_D = 512
_NW = 32
_NQ = _D // 16
_XW = _N * _D


def _active_sets():
    mask2d = np.zeros((_N, _N), dtype=bool)
    mask2d[np.arange(_N), np.arange(_N)] = True
    stride, offset = 1, 0
    for c in _POOLING_COUNTS:
        for _ in range(c):
            offset += stride
            i = np.arange(0, _N - offset, stride)
            mask2d[i, i + offset] = True
        stride *= 2
    return mask2d


def _build_static():
    mask2d = _active_sets()
    entries, starts = [], [0]
    for i in range(_N):
        act = [j | (128 if j == i else 0) for j in range(_N) if mask2d[i, j]]
        prev = [j for j in range(_N) if i >= 2 and mask2d[i - 2, j]]
        stale = [j for j in prev if not mask2d[i, j]]
        entries.extend(act)
        entries.extend(j | 64 for j in stale)
        starts.append(len(entries))
    entries.extend([64 + 63] * 16)
    starts.extend([starts[-1]] * 15)
    return (mask2d,
            np.array(entries, dtype=np.int32),
            np.array(starts, dtype=np.int32))


_MASK2D_NP, _ENTRIES_NP, _STARTS_NP = _build_static()


def _sc_body(x_hbm, tbl_hbm, st_hbm, out_hbm, xtile, tblv, stv, buf0, buf1,
             sem0, sem1):
    cid = lax.axis_index("c")
    sid = lax.axis_index("s")
    wid = sid * 2 + cid

    pltpu.sync_copy(tbl_hbm, tblv)
    pltpu.sync_copy(st_hbm, stv)
    pltpu.sync_copy(x_hbm.at[pl.ds(wid * _XW, _XW)], xtile.at[pl.ds(0, _XW)])

    zero = jnp.zeros((16,), jnp.float32)

    @plsc.parallel_loop(0, _D // 16)
    def _zero_pad(q):
        xtile[pl.ds(_XW + q * 16, 16)] = zero

    @plsc.parallel_loop(0, _XW // 16)
    def _zero_body(k):
        j = lax.shift_right_logical(k, 5)
        q = lax.bitwise_and(k, 31)
        buf0[j, pl.ds(q * 16, 16)] = zero
        buf1[j, pl.ds(q * 16, 16)] = zero

    def _process(g, buf, sem):
        @pl.when(g >= 2)
        def _():
            pltpu.make_async_copy(buf, out_hbm.at[0, 0], sem).wait()

        sv = stv[pl.ds(g, 16)]
        e0 = sv[0]
        e1 = sv[1]
        goff = g * _D

        @plsc.parallel_loop(e0, e1)
        def _jobs(e):
            ev = tblv[pl.ds(e, 16)]
            ent = ev[0]
            j = lax.bitwise_and(ent, 63)
            keep = (1 - lax.bitwise_and(
                lax.shift_right_logical(ent, 6), 1)).astype(jnp.float32)
            kv = jnp.broadcast_to(keep, (16,))
            joff = jnp.where(ent >= 128, _XW, j * _D)
            for q in range(_NQ):
                xi = xtile[pl.ds(goff + q * 16, 16)]
                xj = xtile[pl.ds(joff + q * 16, 16)]
                buf[j, pl.ds(q * 16, 16)] = (xi + xj) * kv

        pltpu.async_copy(buf, out_hbm.at[wid, g], sem)

    def _pair(t, carry):
        _process(2 * t, buf0, sem0)
        _process(2 * t + 1, buf1, sem1)
        return carry

    lax.fori_loop(0, _N // 2, _pair, 0)
    pltpu.make_async_copy(buf0, out_hbm.at[0, 0], sem0).wait()
    pltpu.make_async_copy(buf1, out_hbm.at[0, 0], sem1).wait()


_sc_call = pl.kernel(
    _sc_body,
    out_type=jax.ShapeDtypeStruct((_B, _N, _N, _D), jnp.float32),
    mesh=plsc.VectorSubcoreMesh(core_axis_name="c", subcore_axis_name="s"),
    scratch_types=[
        pltpu.VMEM((_XW + _D,), jnp.float32),
        pltpu.VMEM((_ENTRIES_NP.size,), jnp.int32),
        pltpu.VMEM((_STARTS_NP.size,), jnp.int32),
        pltpu.VMEM((_N, _D), jnp.float32),
        pltpu.VMEM((_N, _D), jnp.float32),
        pltpu.SemaphoreType.DMA,
        pltpu.SemaphoreType.DMA,
    ],
    compiler_params=pltpu.CompilerParams(needs_layout_passes=False),
)


def kernel(x):
    B, D, N = x.shape
    xt = jnp.swapaxes(x, 1, 2).reshape(B * N * D)
    out4 = _sc_call(xt, jnp.asarray(_ENTRIES_NP), jnp.asarray(_STARTS_NP))
    map2d = jnp.transpose(out4, (0, 3, 1, 2))
    mask2d = jnp.broadcast_to(
        jnp.asarray(_MASK2D_NP)[None, None, :, :], (B, 1, N, N))
    return (map2d, mask2d)
